# baseline (speedup 1.0000x reference)
import functools

import jax
import jax.numpy as jnp
from jax import lax
from jax.experimental import pallas as pl
from jax.experimental.pallas import tpu as pltpu
from jax.experimental.pallas import tpu_sc as plsc

B, C, D = 16384, 100, 64
NC, NS, L = 2, 16, 16
NW = NC * NS
NSLAB = C * D // 8
SPW = NSLAB // NW
PARTS = 16
BW = B // PARTS
NCHUNK = SPW * PARTS
NBUF = 8
PF = NBUF - 2
NGRP = NCHUNK // NBUF
KG = BW // L

_mesh = plsc.VectorSubcoreMesh(core_axis_name="c", subcore_axis_name="s")


@functools.partial(
    pl.kernel,
    mesh=_mesh,
    out_type=jax.ShapeDtypeStruct((C, D, B), jnp.float32),
    scratch_types=[
        pltpu.VMEM((C, D), jnp.float32),
        pltpu.VMEM((D, L), jnp.float32),
        pltpu.VMEM((NBUF, 8, BW), jnp.float32),
        pltpu.SemaphoreType.DMA((NBUF,)),
        pltpu.SemaphoreType.DMA((NBUF,)),
    ],
    compiler_params=pltpu.CompilerParams(use_tc_tiling_on_sc=True),
)
def _col_add(x_hbm, t_hbm, o_hbm, tbuf, texp, bufs, sin, sout):
    wid = lax.axis_index("s") * NC + lax.axis_index("c")
    s0 = wid * SPW
    pltpu.sync_copy(t_hbm, tbuf)

    def chunk_coords(i):
        s = s0 + i // PARTS
        return s // 8, s % 8, (i % PARTS) * BW

    def start_in(i, b):
        c, dh, boff = chunk_coords(i)
        pltpu.async_copy(
            x_hbm.at[c, pl.ds(8 * dh, 8), pl.ds(boff, BW)],
            bufs.at[b], sin.at[b])

    def wait_in(b):
        pltpu.make_async_copy(
            x_hbm.at[0, pl.ds(0, 8), pl.ds(0, BW)], bufs.at[b],
            sin.at[b]).wait()

    def start_out(i, b):
        c, dh, boff = chunk_coords(i)
        pltpu.async_copy(
            bufs.at[b], o_hbm.at[c, pl.ds(8 * dh, 8), pl.ds(boff, BW)],
            sout.at[b])

    def wait_out(b):
        pltpu.make_async_copy(
            bufs.at[b], o_hbm.at[0, pl.ds(0, 8), pl.ds(0, BW)],
            sout.at[b]).wait()

    def build_texp(c):
        for g in range(D // L):
            tv = tbuf[c, pl.ds(g * L, L)]
            for j in range(L):
                texp[g * L + j, :] = jnp.broadcast_to(tv[j], (L,))

    def compute(i, b):
        c, dh, _ = chunk_coords(i)
        s = s0 + i // PARTS
        new_c = jnp.logical_and(i % PARTS == 0, s % 8 == 0)

        @pl.when(jnp.logical_or(i == 0, new_c))
        def _():
            build_texp(c)

        for dl in range(8):
            t = texp[8 * dh + dl, :]

            @plsc.parallel_loop(0, KG, unroll=4)
            def _(k):
                bufs[b, dl, pl.ds(k * L, L)] += t

    for b in range(PF):
        start_in(b, b)

    def group(g, carry):
        for b in range(NBUF):
            i = g * NBUF + b
            bp = (b + PF) % NBUF

            @pl.when(i + PF < NCHUNK)
            def _():
                @pl.when(i >= 2)
                def _():
                    wait_out(bp)
                start_in(i + PF, bp)

            wait_in(b)
            compute(i, b)
            start_out(i, b)
        return carry

    lax.fori_loop(0, NGRP, group, 0)
    for b in range(NBUF):
        wait_out(b)


def kernel(inputs, table):
    out_t = _col_add(jnp.transpose(inputs, (1, 2, 0)), table)
    return jnp.transpose(out_t, (2, 0, 1))


# speedup vs baseline: 1.0300x; 1.0300x over previous
import functools

import jax
import jax.numpy as jnp
from jax import lax
from jax.experimental import pallas as pl
from jax.experimental.pallas import tpu as pltpu
from jax.experimental.pallas import tpu_sc as plsc

B, C, D = 16384, 100, 64
NC, NS, L = 2, 16, 16
NW = NC * NS
BPW = B // NW
HD = D // 2
NCHUNK = C * 2
NBUF = 6
PF = NBUF - 2
NGRP = NCHUNK // NBUF
KG = BPW // L

_mesh = plsc.VectorSubcoreMesh(core_axis_name="c", subcore_axis_name="s")


@functools.partial(
    pl.kernel,
    mesh=_mesh,
    out_type=jax.ShapeDtypeStruct((C, D, B), jnp.float32),
    scratch_types=[
        pltpu.VMEM((C, D), jnp.float32),
        pltpu.VMEM((HD, L), jnp.float32),
        pltpu.VMEM((NBUF, HD, BPW), jnp.float32),
        pltpu.SemaphoreType.DMA((NBUF,)),
        pltpu.SemaphoreType.DMA((NBUF,)),
    ],
    compiler_params=pltpu.CompilerParams(use_tc_tiling_on_sc=True),
)
def _col_add(x_hbm, t_hbm, o_hbm, tbuf, texp, bufs, sin, sout):
    wid = lax.axis_index("s") * NC + lax.axis_index("c")
    b0 = wid * BPW
    pltpu.sync_copy(t_hbm, tbuf)

    def start_in(i, b):
        c, h = i // 2, (i % 2) * HD
        pltpu.async_copy(
            x_hbm.at[c, pl.ds(h, HD), pl.ds(b0, BPW)], bufs.at[b], sin.at[b])

    def wait_in(b):
        pltpu.make_async_copy(
            x_hbm.at[0, pl.ds(0, HD), pl.ds(b0, BPW)], bufs.at[b],
            sin.at[b]).wait()

    def start_out(i, b):
        c, h = i // 2, (i % 2) * HD
        pltpu.async_copy(
            bufs.at[b], o_hbm.at[c, pl.ds(h, HD), pl.ds(b0, BPW)], sout.at[b])

    def wait_out(b):
        pltpu.make_async_copy(
            bufs.at[b], o_hbm.at[0, pl.ds(0, HD), pl.ds(0, BPW)],
            sout.at[b]).wait()

    def compute(i, b):
        c, h = i // 2, (i % 2) * HD
        for g in range(HD // L):
            tv = tbuf[c, pl.ds(h + g * L, L)]
            for j in range(L):
                texp[g * L + j, :] = jnp.broadcast_to(tv[j], (L,))

        @plsc.parallel_loop(0, HD)
        def _(d):
            t = texp[d, :]
            for k in range(KG):
                bufs[b, d, pl.ds(k * L, L)] += t

    for b in range(PF):
        start_in(b, b)

    def step(i, b):
        bp = (b + PF) % NBUF

        @pl.when(i + PF < NCHUNK)
        def _():
            @pl.when(i >= 2)
            def _():
                wait_out(bp)
            start_in(i + PF, bp)

        wait_in(b)
        compute(i, b)
        start_out(i, b)

    def group(g, carry):
        for b in range(NBUF):
            step(g * NBUF + b, b)
        return carry

    lax.fori_loop(0, NGRP, group, 0)
    for r in range(NGRP * NBUF, NCHUNK):
        step(r, r % NBUF)
    for b in range(NBUF):
        wait_out(b)


def kernel(inputs, table):
    out_t = _col_add(jnp.transpose(inputs, (1, 2, 0)), table)
    return jnp.transpose(out_t, (2, 0, 1))


# speedup vs baseline: 1.0310x; 1.0010x over previous
import functools

import jax
import jax.numpy as jnp
from jax import lax
from jax.experimental import pallas as pl
from jax.experimental.pallas import tpu as pltpu
from jax.experimental.pallas import tpu_sc as plsc

B, C, D = 16384, 100, 64
NC, NS, L = 2, 16, 16
NW = NC * NS
BPW = B // NW
HD = D // 2
NCHUNK = C * 2
NBUF = 6
PF = NBUF - 2
NGRP = NCHUNK // NBUF
KG = BPW // L

_mesh = plsc.VectorSubcoreMesh(core_axis_name="c", subcore_axis_name="s")


@functools.partial(
    pl.kernel,
    mesh=_mesh,
    out_type=jax.ShapeDtypeStruct((C, D, B), jnp.float32),
    scratch_types=[
        pltpu.VMEM((C, D), jnp.float32),
        pltpu.VMEM((HD, L), jnp.float32),
        pltpu.VMEM((NBUF, HD, BPW), jnp.float32),
        pltpu.SemaphoreType.DMA((NBUF,)),
        pltpu.SemaphoreType.DMA((NBUF,)),
    ],
    compiler_params=pltpu.CompilerParams(
        use_tc_tiling_on_sc=True, skip_device_barrier=True),
)
def _col_add(x_hbm, t_hbm, o_hbm, tbuf, texp, bufs, sin, sout):
    wid = lax.axis_index("s") * NC + lax.axis_index("c")
    b0 = wid * BPW
    pltpu.sync_copy(t_hbm, tbuf)

    def start_in(i, b):
        c, h = i // 2, (i % 2) * HD
        pltpu.async_copy(
            x_hbm.at[c, pl.ds(h, HD), pl.ds(b0, BPW)], bufs.at[b], sin.at[b])

    def wait_in(b):
        pltpu.make_async_copy(
            x_hbm.at[0, pl.ds(0, HD), pl.ds(b0, BPW)], bufs.at[b],
            sin.at[b]).wait()

    def start_out(i, b):
        c, h = i // 2, (i % 2) * HD
        pltpu.async_copy(
            bufs.at[b], o_hbm.at[c, pl.ds(h, HD), pl.ds(b0, BPW)], sout.at[b])

    def wait_out(b):
        pltpu.make_async_copy(
            bufs.at[b], o_hbm.at[0, pl.ds(0, HD), pl.ds(0, BPW)],
            sout.at[b]).wait()

    def compute(i, b):
        c, h = i // 2, (i % 2) * HD
        for g in range(HD // L):
            tv = tbuf[c, pl.ds(h + g * L, L)]
            for j in range(L):
                texp[g * L + j, :] = jnp.broadcast_to(tv[j], (L,))

        @plsc.parallel_loop(0, HD)
        def _(d):
            t = texp[d, :]
            for k in range(KG):
                bufs[b, d, pl.ds(k * L, L)] += t

    for b in range(PF):
        start_in(b, b)

    def step(i, b):
        bp = (b + PF) % NBUF

        @pl.when(i + PF < NCHUNK)
        def _():
            @pl.when(i >= 2)
            def _():
                wait_out(bp)
            start_in(i + PF, bp)

        wait_in(b)
        compute(i, b)
        start_out(i, b)

    def group(g, carry):
        for b in range(NBUF):
            step(g * NBUF + b, b)
        return carry

    lax.fori_loop(0, NGRP, group, 0)
    for r in range(NGRP * NBUF, NCHUNK):
        step(r, r % NBUF)
    for b in range(NBUF):
        wait_out(b)


def kernel(inputs, table):
    out_t = _col_add(jnp.transpose(inputs, (1, 2, 0)), table)
    return jnp.transpose(out_t, (2, 0, 1))
